# baseline (speedup 1.0000x reference)
import functools

import jax
import jax.numpy as jnp
from jax import lax
from jax.experimental import pallas as pl
from jax.experimental.pallas import tpu as pltpu
from jax.experimental.pallas import tpu_sc as plsc

N = 10000
D = 128
E = 320000

NC = 2
NS = 16
NW = NC * NS

NP = 10240
RPT = NP // NS
EP = 327680
EW = EP // NW
EROWS = EP // 128
ERPW = EW // 128
IBR = 16

_mesh = plsc.VectorSubcoreMesh(core_axis_name="c", subcore_axis_name="s")


def _sc_degrees_body(src2, dst2, ones_s_hbm, ones_d_hbm, zeros_hbm, deg_out,
                     sidx, didx, ones_s, ones_d, acc, sem, semi):
  c = lax.axis_index("c")
  s = lax.axis_index("s")
  wid = c * NS + s
  for k in range(RPT // 128):
    pltpu.sync_copy(zeros_hbm, acc.at[pl.ds(s * RPT + k * 128, 128)])
  pltpu.sync_copy(ones_s_hbm, ones_s)
  pltpu.sync_copy(ones_d_hbm, ones_d)
  ebase = wid * ERPW
  pltpu.sync_copy(src2.at[pl.ds(ebase, IBR)], sidx.at[0])
  pltpu.sync_copy(dst2.at[pl.ds(ebase, IBR)], didx.at[0])
  plsc.subcore_barrier()

  def step(ib, carry):
    blk = lax.div(ib, IBR)
    pb = lax.rem(blk, 2)
    r = lax.rem(ib, IBR)

    @pl.when(jnp.logical_and(r == 3, ib + IBR < ERPW))
    def _():
      nb = (blk + 1) * IBR
      pltpu.async_copy(src2.at[pl.ds(ebase + nb, IBR)], sidx.at[1 - pb],
                       semi.at[0])
      pltpu.async_copy(dst2.at[pl.ds(ebase + nb, IBR)], didx.at[1 - pb],
                       semi.at[1])

    @pl.when(jnp.logical_and(r == IBR - 1, ib + IBR < ERPW))
    def _():
      pltpu.make_async_copy(src2.at[pl.ds(ebase, IBR)], sidx.at[0],
                            semi.at[0]).wait()
      pltpu.make_async_copy(dst2.at[pl.ds(ebase, IBR)], didx.at[0],
                            semi.at[1]).wait()

    @pl.when(ib >= 2)
    def _():
      pltpu.make_async_copy(ones_s_hbm, ones_s, sem).wait()
      pltpu.make_async_copy(ones_s_hbm, ones_s, sem).wait()

    pltpu.async_copy(ones_s, acc.at[sidx.at[pb, r]], sem, add=True)
    pltpu.async_copy(ones_d, acc.at[didx.at[pb, r]], sem, add=True)
    return carry

  lax.fori_loop(0, ERPW, step, 0)
  for _ in range(4):
    pltpu.make_async_copy(ones_s_hbm, ones_s, sem).wait()
  plsc.subcore_barrier()
  pltpu.sync_copy(acc.at[pl.ds(s * RPT, RPT)], deg_out.at[c, pl.ds(s * RPT, RPT)])


_sc_degrees = pl.kernel(
    _sc_degrees_body,
    out_type=jax.ShapeDtypeStruct((NC, NP, D), jnp.float32),
    mesh=_mesh,
    scratch_types=[
        pltpu.VMEM((2, IBR, 128), jnp.int32),
        pltpu.VMEM((2, IBR, 128), jnp.int32),
        pltpu.VMEM((128, D), jnp.float32),
        pltpu.VMEM((128, D), jnp.float32),
        pltpu.VMEM_SHARED((NP, D), jnp.float32),
        pltpu.SemaphoreType.DMA,
        pltpu.SemaphoreType.DMA((2,)),
    ],
)


R0SPLIT = 160
R1SPLIT = (EROWS - NS * R0SPLIT) // NS


def _sc_msgpass_body(h, src2, dst2, zeros_hbm, out, sidx, didx, rows, acc,
                     semg, sems, semi):
  c = lax.axis_index("c")
  s = lax.axis_index("s")
  for k in range(RPT // 128):
    pltpu.sync_copy(zeros_hbm, acc.at[pl.ds(s * RPT + k * 128, 128)])
  nrows = jnp.where(c == 0, R0SPLIT, R1SPLIT)
  ebase = jnp.minimum(
      jnp.where(c == 0, s * R0SPLIT, NS * R0SPLIT + s * R1SPLIT),
      EROWS - IBR)

  @pl.when(nrows > 0)
  def _():
    pltpu.sync_copy(src2.at[pl.ds(ebase, IBR)], sidx.at[0])
    pltpu.sync_copy(dst2.at[pl.ds(ebase, IBR)], didx.at[0])

  plsc.subcore_barrier()

  def gather_chunk(pb_, r_, buf):
    for j in range(4):
      pltpu.async_copy(h.at[sidx.at[pb_, r_, pl.ds(32 * j, 32)]],
                       rows.at[buf, pl.ds(32 * j, 32)], semg.at[buf])

  @pl.when(nrows > 0)
  def _():
    gather_chunk(0, 0, 0)

  def step(ib, carry):
    p = lax.rem(ib, 2)
    q = 1 - p
    blk = lax.div(ib, IBR)
    pb = lax.rem(blk, 2)
    r = lax.rem(ib, IBR)

    @pl.when(ib >= 1)
    def _():
      pltpu.make_async_copy(h.at[sidx.at[0, 0]], rows.at[q], sems.at[q]).wait()

    @pl.when(jnp.logical_and(r == 2, ib + IBR < nrows))
    def _():
      nb = (blk + 1) * IBR
      pltpu.async_copy(src2.at[pl.ds(ebase + nb, IBR)], sidx.at[1 - pb],
                       semi.at[0])
      pltpu.async_copy(dst2.at[pl.ds(ebase + nb, IBR)], didx.at[1 - pb],
                       semi.at[1])

    @pl.when(jnp.logical_and(r == IBR - 1, ib + IBR < nrows))
    def _():
      pltpu.make_async_copy(src2.at[pl.ds(ebase, IBR)], sidx.at[0],
                            semi.at[0]).wait()
      pltpu.make_async_copy(dst2.at[pl.ds(ebase, IBR)], didx.at[0],
                            semi.at[1]).wait()

    @pl.when(ib + 1 < nrows)
    def _():
      nxt = ib + 1
      gather_chunk(lax.rem(lax.div(nxt, IBR), 2), lax.rem(nxt, IBR), q)

    pltpu.make_async_copy(h.at[sidx.at[0, 0]], rows.at[p], semg.at[p]).wait()
    pltpu.async_copy(rows.at[p], acc.at[didx.at[pb, r]], sems.at[p], add=True)
    return carry

  lax.fori_loop(0, nrows, step, 0)

  @pl.when(nrows > 0)
  def _():
    pltpu.make_async_copy(h.at[sidx.at[0, 0]], rows.at[1], sems.at[1]).wait()

  plsc.subcore_barrier()
  pltpu.sync_copy(acc.at[pl.ds(s * RPT, RPT)], out.at[c, pl.ds(s * RPT, RPT)])


_sc_msgpass = pl.kernel(
    _sc_msgpass_body,
    out_type=jax.ShapeDtypeStruct((NC, NP, D), jnp.float32),
    mesh=_mesh,
    scratch_types=[
        pltpu.VMEM((2, IBR, 128), jnp.int32),
        pltpu.VMEM((2, IBR, 128), jnp.int32),
        pltpu.VMEM((2, 128, D), jnp.float32),
        pltpu.VMEM_SHARED((NP, D), jnp.float32),
        pltpu.SemaphoreType.DMA((2,)),
        pltpu.SemaphoreType.DMA((2,)),
        pltpu.SemaphoreType.DMA((2,)),
    ],
)


RB = 1024


def _tc_prep_body(x_ref, dg0, dg1, h_ref, ns_ref, nd_ref):
  deg = dg0[...] + dg1[...]
  dsrc = jnp.maximum(deg[:, 0:1], 1.0)
  ddst = jnp.maximum(deg[:, 64:65], 1.0)
  ns = lax.rsqrt(dsrc)
  nd = lax.rsqrt(ddst)
  ns_ref[...] = jnp.broadcast_to(ns, (RB, 16))
  nd_ref[...] = jnp.broadcast_to(nd, (RB, 16))
  h_ref[...] = x_ref[...] * ns


def _tc_prep(xpad, dg0, dg1):
  grid = (NP // RB,)
  row = pl.BlockSpec((RB, D), lambda i: (i, 0))
  deg = pl.BlockSpec((RB, 16), lambda i: (i, 0))
  return pl.pallas_call(
      _tc_prep_body,
      grid=grid,
      in_specs=[row, row, row],
      out_specs=[row, deg, deg],
      out_shape=[
          jax.ShapeDtypeStruct((NP, D), jnp.float32),
          jax.ShapeDtypeStruct((NP, 16), jnp.float32),
          jax.ShapeDtypeStruct((NP, 16), jnp.float32),
      ],
  )(xpad, dg0, dg1)


def _tc_layer_body(p0, p1, nd, ns, w_ref, b_ref, o_ref, *, final):
  agg = (p0[...] + p1[...]) * nd[...][:, 0:1]
  z = jnp.dot(agg, w_ref[...], preferred_element_type=jnp.float32) + b_ref[...]
  if final:
    o_ref[...] = z
  else:
    o_ref[...] = jnp.maximum(z, 0.0) * ns[...][:, 0:1]


def _tc_layer(p0, p1, nd, ns, w, b, final):
  grid = (NP // RB,)
  row = pl.BlockSpec((RB, D), lambda i: (i, 0))
  deg = pl.BlockSpec((RB, 16), lambda i: (i, 0))
  full = pl.BlockSpec((D, D), lambda i: (0, 0))
  bias = pl.BlockSpec((1, D), lambda i: (0, 0))
  odtype = jnp.float32
  return pl.pallas_call(
      functools.partial(_tc_layer_body, final=final),
      grid=grid,
      in_specs=[row, row, deg, deg, full, bias],
      out_specs=row,
      out_shape=jax.ShapeDtypeStruct((NP, D), odtype),
  )(p0, p1, nd, ns, w, b)


def kernel(x, W1, b1, W2, b2, W3, b3, edge_index):
  src = edge_index[0]
  dst = edge_index[1]
  pad = jnp.full((EP - E,), NP - 1, dtype=jnp.int32)
  src2 = jnp.concatenate([src, pad]).reshape(EROWS, 128)
  dst2 = jnp.concatenate([dst, pad]).reshape(EROWS, 128)
  xpad = jnp.pad(x, ((0, NP - N), (0, 0)))

  zeros128 = jnp.zeros((128, D), jnp.float32)
  lane = jnp.arange(D, dtype=jnp.int32)
  ones_s = jnp.broadcast_to((lane < 64).astype(jnp.float32), (128, D))
  ones_d = jnp.broadcast_to((lane >= 64).astype(jnp.float32), (128, D))

  deg = _sc_degrees(src2, dst2, ones_s, ones_d, zeros128)
  h, ns, nd = _tc_prep(xpad, deg[0], deg[1])

  p = _sc_msgpass(h, src2, dst2, zeros128)
  h = _tc_layer(p[0], p[1], nd, ns, W1, b1.reshape(1, D), final=False)
  p = _sc_msgpass(h, src2, dst2, zeros128)
  h = _tc_layer(p[0], p[1], nd, ns, W2, b2.reshape(1, D), final=False)
  p = _sc_msgpass(h, src2, dst2, zeros128)
  out = _tc_layer(p[0], p[1], nd, ns, W3, b3.reshape(1, D), final=True)
  return out[:N]


# speedup vs baseline: 1.1920x; 1.1920x over previous
import functools

import jax
import jax.numpy as jnp
from jax import lax
from jax.experimental import pallas as pl
from jax.experimental.pallas import tpu as pltpu
from jax.experimental.pallas import tpu_sc as plsc

N = 10000
D = 128
E = 320000

NC = 2
NS = 16
NW = NC * NS

NP = 10240
RPT = NP // NS
EP = 327680
EW = EP // NW
EROWS = EP // 128
ERPW = EW // 128
IBR = 16

_mesh = plsc.VectorSubcoreMesh(core_axis_name="c", subcore_axis_name="s")


def _sc_degrees_body(src2, dst2, ones_s_hbm, ones_d_hbm, zeros_hbm, deg_out,
                     sidx, didx, ones_s, ones_d, acc, sem, semi):
  c = lax.axis_index("c")
  s = lax.axis_index("s")
  wid = c * NS + s
  for k in range(RPT // 128):
    pltpu.sync_copy(zeros_hbm, acc.at[pl.ds(s * RPT + k * 128, 128)])
  pltpu.sync_copy(ones_s_hbm, ones_s)
  pltpu.sync_copy(ones_d_hbm, ones_d)
  ebase = wid * ERPW
  pltpu.sync_copy(src2.at[pl.ds(ebase, IBR)], sidx.at[0])
  pltpu.sync_copy(dst2.at[pl.ds(ebase, IBR)], didx.at[0])
  plsc.subcore_barrier()

  def step(ib, carry):
    blk = lax.div(ib, IBR)
    pb = lax.rem(blk, 2)
    r = lax.rem(ib, IBR)

    @pl.when(jnp.logical_and(r == 3, ib + IBR < ERPW))
    def _():
      nb = (blk + 1) * IBR
      pltpu.async_copy(src2.at[pl.ds(ebase + nb, IBR)], sidx.at[1 - pb],
                       semi.at[0])
      pltpu.async_copy(dst2.at[pl.ds(ebase + nb, IBR)], didx.at[1 - pb],
                       semi.at[1])

    @pl.when(jnp.logical_and(r == IBR - 1, ib + IBR < ERPW))
    def _():
      pltpu.make_async_copy(src2.at[pl.ds(ebase, IBR)], sidx.at[0],
                            semi.at[0]).wait()
      pltpu.make_async_copy(dst2.at[pl.ds(ebase, IBR)], didx.at[0],
                            semi.at[1]).wait()

    @pl.when(ib >= 2)
    def _():
      pltpu.make_async_copy(ones_s_hbm, ones_s, sem).wait()
      pltpu.make_async_copy(ones_s_hbm, ones_s, sem).wait()

    pltpu.async_copy(ones_s, acc.at[sidx.at[pb, r]], sem, add=True)
    pltpu.async_copy(ones_d, acc.at[didx.at[pb, r]], sem, add=True)
    return carry

  lax.fori_loop(0, ERPW, step, 0)
  for _ in range(4):
    pltpu.make_async_copy(ones_s_hbm, ones_s, sem).wait()
  plsc.subcore_barrier()
  pltpu.sync_copy(acc.at[pl.ds(s * RPT, RPT)], deg_out.at[c, pl.ds(s * RPT, RPT)])


_sc_degrees = pl.kernel(
    _sc_degrees_body,
    out_type=jax.ShapeDtypeStruct((NC, NP, D), jnp.float32),
    mesh=_mesh,
    scratch_types=[
        pltpu.VMEM((2, IBR, 128), jnp.int32),
        pltpu.VMEM((2, IBR, 128), jnp.int32),
        pltpu.VMEM((128, D), jnp.float32),
        pltpu.VMEM((128, D), jnp.float32),
        pltpu.VMEM_SHARED((NP, D), jnp.float32),
        pltpu.SemaphoreType.DMA,
        pltpu.SemaphoreType.DMA((2,)),
    ],
)


R0SPLIT = 128
R1SPLIT = (EROWS - NS * R0SPLIT) // NS


def _sc_msgpass_body(h, src2, dst2, zeros_hbm, out, sidx, didx, rows, acc,
                     semg, sems, semi):
  c = lax.axis_index("c")
  s = lax.axis_index("s")
  for k in range(RPT // 128):
    pltpu.sync_copy(zeros_hbm, acc.at[pl.ds(s * RPT + k * 128, 128)])
  nrows = jnp.where(c == 0, R0SPLIT, R1SPLIT)
  ebase = jnp.minimum(
      jnp.where(c == 0, s * R0SPLIT, NS * R0SPLIT + s * R1SPLIT),
      EROWS - IBR)

  @pl.when(nrows > 0)
  def _():
    pltpu.sync_copy(src2.at[pl.ds(ebase, IBR)], sidx.at[0])
    pltpu.sync_copy(dst2.at[pl.ds(ebase, IBR)], didx.at[0])

  plsc.subcore_barrier()

  def gather_chunk(pb_, r_, buf):
    for j in range(4):
      pltpu.async_copy(h.at[sidx.at[pb_, r_, pl.ds(32 * j, 32)]],
                       rows.at[buf, pl.ds(32 * j, 32)], semg.at[buf])

  @pl.when(nrows > 0)
  def _():
    gather_chunk(0, 0, 0)

  def step(ib, carry):
    p = lax.rem(ib, 2)
    q = 1 - p
    blk = lax.div(ib, IBR)
    pb = lax.rem(blk, 2)
    r = lax.rem(ib, IBR)

    @pl.when(ib >= 1)
    def _():
      pltpu.make_async_copy(h.at[sidx.at[0, 0]], rows.at[q], sems.at[q]).wait()

    @pl.when(jnp.logical_and(r == 2, ib + IBR < nrows))
    def _():
      nb = (blk + 1) * IBR
      pltpu.async_copy(src2.at[pl.ds(ebase + nb, IBR)], sidx.at[1 - pb],
                       semi.at[0])
      pltpu.async_copy(dst2.at[pl.ds(ebase + nb, IBR)], didx.at[1 - pb],
                       semi.at[1])

    @pl.when(jnp.logical_and(r == IBR - 1, ib + IBR < nrows))
    def _():
      pltpu.make_async_copy(src2.at[pl.ds(ebase, IBR)], sidx.at[0],
                            semi.at[0]).wait()
      pltpu.make_async_copy(dst2.at[pl.ds(ebase, IBR)], didx.at[0],
                            semi.at[1]).wait()

    @pl.when(ib + 1 < nrows)
    def _():
      nxt = ib + 1
      gather_chunk(lax.rem(lax.div(nxt, IBR), 2), lax.rem(nxt, IBR), q)

    pltpu.make_async_copy(h.at[sidx.at[0, 0]], rows.at[p], semg.at[p]).wait()
    pltpu.async_copy(rows.at[p], acc.at[didx.at[pb, r]], sems.at[p], add=True)
    return carry

  lax.fori_loop(0, nrows, step, 0)

  @pl.when(nrows > 0)
  def _():
    pltpu.make_async_copy(h.at[sidx.at[0, 0]], rows.at[1], sems.at[1]).wait()

  plsc.subcore_barrier()
  pltpu.sync_copy(acc.at[pl.ds(s * RPT, RPT)], out.at[c, pl.ds(s * RPT, RPT)])


_sc_msgpass = pl.kernel(
    _sc_msgpass_body,
    out_type=jax.ShapeDtypeStruct((NC, NP, D), jnp.float32),
    mesh=_mesh,
    scratch_types=[
        pltpu.VMEM((2, IBR, 128), jnp.int32),
        pltpu.VMEM((2, IBR, 128), jnp.int32),
        pltpu.VMEM((2, 128, D), jnp.float32),
        pltpu.VMEM_SHARED((NP, D), jnp.float32),
        pltpu.SemaphoreType.DMA((2,)),
        pltpu.SemaphoreType.DMA((2,)),
        pltpu.SemaphoreType.DMA((2,)),
    ],
)


RB = 1024


def _tc_prep_body(x_ref, dg0, dg1, h_ref, ns_ref, nd_ref):
  deg = dg0[...] + dg1[...]
  dsrc = jnp.maximum(deg[:, 0:1], 1.0)
  ddst = jnp.maximum(deg[:, 64:65], 1.0)
  ns = lax.rsqrt(dsrc)
  nd = lax.rsqrt(ddst)
  ns_ref[...] = jnp.broadcast_to(ns, (RB, 16))
  nd_ref[...] = jnp.broadcast_to(nd, (RB, 16))
  h_ref[...] = x_ref[...] * ns


def _tc_prep(xpad, dg0, dg1):
  grid = (NP // RB,)
  row = pl.BlockSpec((RB, D), lambda i: (i, 0))
  deg = pl.BlockSpec((RB, 16), lambda i: (i, 0))
  return pl.pallas_call(
      _tc_prep_body,
      grid=grid,
      in_specs=[row, row, row],
      out_specs=[row, deg, deg],
      out_shape=[
          jax.ShapeDtypeStruct((NP, D), jnp.float32),
          jax.ShapeDtypeStruct((NP, 16), jnp.float32),
          jax.ShapeDtypeStruct((NP, 16), jnp.float32),
      ],
  )(xpad, dg0, dg1)


def _tc_layer_body(p0, p1, nd, ns, w_ref, b_ref, o_ref, *, final):
  agg = (p0[...] + p1[...]) * nd[...][:, 0:1]
  z = jnp.dot(agg, w_ref[...], preferred_element_type=jnp.float32) + b_ref[...]
  if final:
    o_ref[...] = z
  else:
    o_ref[...] = jnp.maximum(z, 0.0) * ns[...][:, 0:1]


def _tc_layer(p0, p1, nd, ns, w, b, final):
  grid = (NP // RB,)
  row = pl.BlockSpec((RB, D), lambda i: (i, 0))
  deg = pl.BlockSpec((RB, 16), lambda i: (i, 0))
  full = pl.BlockSpec((D, D), lambda i: (0, 0))
  bias = pl.BlockSpec((1, D), lambda i: (0, 0))
  odtype = jnp.float32
  return pl.pallas_call(
      functools.partial(_tc_layer_body, final=final),
      grid=grid,
      in_specs=[row, row, deg, deg, full, bias],
      out_specs=row,
      out_shape=jax.ShapeDtypeStruct((NP, D), odtype),
  )(p0, p1, nd, ns, w, b)


def kernel(x, W1, b1, W2, b2, W3, b3, edge_index):
  src = edge_index[0]
  dst = edge_index[1]
  pad = jnp.full((EP - E,), NP - 1, dtype=jnp.int32)
  src2 = jnp.concatenate([src, pad]).reshape(EROWS, 128)
  dst2 = jnp.concatenate([dst, pad]).reshape(EROWS, 128)
  xpad = jnp.pad(x, ((0, NP - N), (0, 0)))

  zeros128 = jnp.zeros((128, D), jnp.float32)
  lane = jnp.arange(D, dtype=jnp.int32)
  ones_s = jnp.broadcast_to((lane < 64).astype(jnp.float32), (128, D))
  ones_d = jnp.broadcast_to((lane >= 64).astype(jnp.float32), (128, D))

  deg = _sc_degrees(src2, dst2, ones_s, ones_d, zeros128)
  h, ns, nd = _tc_prep(xpad, deg[0], deg[1])

  p = _sc_msgpass(h, src2, dst2, zeros128)
  h = _tc_layer(p[0], p[1], nd, ns, W1, b1.reshape(1, D), final=False)
  p = _sc_msgpass(h, src2, dst2, zeros128)
  h = _tc_layer(p[0], p[1], nd, ns, W2, b2.reshape(1, D), final=False)
  p = _sc_msgpass(h, src2, dst2, zeros128)
  out = _tc_layer(p[0], p[1], nd, ns, W3, b3.reshape(1, D), final=True)
  return out[:N]
